# Initial kernel scaffold; baseline (speedup 1.0000x reference)
import jax
import jax.numpy as jnp
from jax.experimental import pallas as pl


def kernel(x, weight, bias):
    raise NotImplementedError("write your pallas kernel here")


# speedup vs baseline: 1.0163x; 1.0163x over previous
import functools

import jax
import jax.numpy as jnp
from jax.experimental import pallas as pl
from jax.experimental.pallas import tpu as pltpu

_TOPK = 8


def _gate_body(x_ref, wt_ref, bias_ref, w_out_ref, i_out_ref, *, n_experts):
    x = x_ref[...]
    wt = wt_ref[...]
    scores = jnp.dot(x, wt, preferred_element_type=jnp.float32)
    scores = scores + bias_ref[...]

    idx = jax.lax.broadcasted_iota(jnp.int32, scores.shape, 1)
    vals = scores
    top_vs = []
    top_is = []
    for _ in range(_TOPK):
        m = jnp.max(vals, axis=1, keepdims=True)
        am = jnp.min(jnp.where(vals == m, idx, n_experts), axis=1,
                     keepdims=True)
        top_vs.append(m)
        top_is.append(am)
        vals = jnp.where(idx == am, -jnp.inf, vals)

    tv = jnp.concatenate(top_vs, axis=1)
    ti = jnp.concatenate(top_is, axis=1)
    e = jnp.exp(tv - tv[:, :1])
    w = e / jnp.sum(e, axis=1, keepdims=True)
    w_out_ref[...] = w
    i_out_ref[...] = ti.astype(jnp.int32)


@functools.partial(jax.jit, static_argnames=("block_rows",))
def _moe_gate(x, weight, bias, block_rows=512):
    n_rows, k = x.shape
    n_experts = weight.shape[0]
    wt = weight.T
    bias2d = bias.reshape(1, n_experts)

    grid = (n_rows // block_rows,)
    out_w, out_i = pl.pallas_call(
        functools.partial(_gate_body, n_experts=n_experts),
        grid=grid,
        in_specs=[
            pl.BlockSpec((block_rows, k), lambda i: (i, 0)),
            pl.BlockSpec((k, n_experts), lambda i: (0, 0)),
            pl.BlockSpec((1, n_experts), lambda i: (0, 0)),
        ],
        out_specs=[
            pl.BlockSpec((block_rows, _TOPK), lambda i: (i, 0)),
            pl.BlockSpec((block_rows, _TOPK), lambda i: (i, 0)),
        ],
        out_shape=[
            jax.ShapeDtypeStruct((n_rows, _TOPK), jnp.float32),
            jax.ShapeDtypeStruct((n_rows, _TOPK), jnp.int32),
        ],
        compiler_params=pltpu.CompilerParams(
            dimension_semantics=("arbitrary",),
        ),
    )(x, wt, bias2d)
    return out_w, out_i


def kernel(x, weight, bias):
    w, i = _moe_gate(x, weight, bias)
    return w.astype(x.dtype), i


# speedup vs baseline: 1.6645x; 1.6378x over previous
import functools

import jax
import jax.numpy as jnp
from jax.experimental import pallas as pl
from jax.experimental.pallas import tpu as pltpu

_TOPK = 8


def _gate_body(x_ref, wt_ref, bias_ref, w_out_ref, i_out_ref, *, n_experts):
    x = x_ref[...]
    wt = wt_ref[...]
    scores = jnp.dot(x, wt, preferred_element_type=jnp.float32)
    scores = scores + bias_ref[...]

    st = scores.T
    idx = jax.lax.broadcasted_iota(jnp.int32, st.shape, 0).astype(jnp.float32)
    vals = st
    top_vs = []
    top_is = []
    for _ in range(_TOPK):
        m = jnp.max(vals, axis=0, keepdims=True)
        am = jnp.min(jnp.where(vals == m, idx, float(n_experts)), axis=0,
                     keepdims=True)
        top_vs.append(m)
        top_is.append(am)
        vals = jnp.where(idx == am, -jnp.inf, vals)

    tv = jnp.concatenate(top_vs, axis=0)
    ti = jnp.concatenate(top_is, axis=0)
    e = jnp.exp(tv - tv[0:1])
    w = e / jnp.sum(e, axis=0, keepdims=True)
    w_out_ref[...] = w
    i_out_ref[...] = ti.astype(jnp.int32)


@functools.partial(jax.jit, static_argnames=("block_rows",))
def _moe_gate(x, weight, bias, block_rows=512):
    n_rows, k = x.shape
    n_experts = weight.shape[0]
    wt = weight.T
    bias2d = bias.reshape(1, n_experts)

    grid = (n_rows // block_rows,)
    out_w, out_i = pl.pallas_call(
        functools.partial(_gate_body, n_experts=n_experts),
        grid=grid,
        in_specs=[
            pl.BlockSpec((block_rows, k), lambda i: (i, 0)),
            pl.BlockSpec((k, n_experts), lambda i: (0, 0)),
            pl.BlockSpec((1, n_experts), lambda i: (0, 0)),
        ],
        out_specs=[
            pl.BlockSpec((_TOPK, block_rows), lambda i: (0, i)),
            pl.BlockSpec((_TOPK, block_rows), lambda i: (0, i)),
        ],
        out_shape=[
            jax.ShapeDtypeStruct((_TOPK, n_rows), jnp.float32),
            jax.ShapeDtypeStruct((_TOPK, n_rows), jnp.int32),
        ],
        compiler_params=pltpu.CompilerParams(
            dimension_semantics=("arbitrary",),
        ),
    )(x, wt, bias2d)
    return out_w.T, out_i.T


def kernel(x, weight, bias):
    w, i = _moe_gate(x, weight, bias)
    return w.astype(x.dtype), i


# speedup vs baseline: 1.6787x; 1.0085x over previous
import functools

import jax
import jax.numpy as jnp
from jax.experimental import pallas as pl
from jax.experimental.pallas import tpu as pltpu

_TOPK = 8


def _gate_body(x_ref, wt_ref, bias_ref, w_out_ref, i_out_ref, *, n_experts):
    x = x_ref[...]
    wt = wt_ref[...]
    scores = jnp.dot(x, wt, preferred_element_type=jnp.float32)
    scores = scores + bias_ref[...]

    st = scores.T
    idx = jax.lax.broadcasted_iota(jnp.int32, st.shape, 0).astype(jnp.float32)
    vals = st
    top_vs = []
    top_is = []
    for _ in range(_TOPK):
        m = jnp.max(vals, axis=0, keepdims=True)
        am = jnp.min(jnp.where(vals == m, idx, float(n_experts)), axis=0,
                     keepdims=True)
        top_vs.append(m)
        top_is.append(am)
        vals = jnp.where(idx == am, -jnp.inf, vals)

    tv = jnp.concatenate(top_vs, axis=0)
    ti = jnp.concatenate(top_is, axis=0)
    e = jnp.exp(tv - tv[0:1])
    w = e / jnp.sum(e, axis=0, keepdims=True)
    w_out_ref[...] = w
    i_out_ref[...] = ti.astype(jnp.int32)


@functools.partial(jax.jit, static_argnames=("block_rows",))
def _moe_gate(x, weight, bias, block_rows=1024):
    n_rows, k = x.shape
    n_experts = weight.shape[0]
    wt = weight.T
    bias2d = bias.reshape(1, n_experts)

    grid = (n_rows // block_rows,)
    out_w, out_i = pl.pallas_call(
        functools.partial(_gate_body, n_experts=n_experts),
        grid=grid,
        in_specs=[
            pl.BlockSpec((block_rows, k), lambda i: (i, 0)),
            pl.BlockSpec((k, n_experts), lambda i: (0, 0)),
            pl.BlockSpec((1, n_experts), lambda i: (0, 0)),
        ],
        out_specs=[
            pl.BlockSpec((_TOPK, block_rows), lambda i: (0, i)),
            pl.BlockSpec((_TOPK, block_rows), lambda i: (0, i)),
        ],
        out_shape=[
            jax.ShapeDtypeStruct((_TOPK, n_rows), jnp.float32),
            jax.ShapeDtypeStruct((_TOPK, n_rows), jnp.int32),
        ],
        compiler_params=pltpu.CompilerParams(
            dimension_semantics=("arbitrary",),
        ),
    )(x, wt, bias2d)
    return out_w.T, out_i.T


def kernel(x, weight, bias):
    w, i = _moe_gate(x, weight, bias)
    return w.astype(x.dtype), i
